# baseline (speedup 1.0000x reference)
import functools

import jax
import jax.numpy as jnp
from jax import lax
from jax.experimental import pallas as pl
from jax.experimental.pallas import tpu as pltpu
from jax.experimental.pallas import tpu_sc as plsc

MODEL_DIM = 768
DIM_FF = 2048
NUM_EXPERTS = 8
TOP_K = 2

BT = 256
LANES = 16


def _gate_body(xf_ref, gw_ref, gb_ref, idx_ref, w_ref):
    x = xf_ref[...]
    logits = jnp.dot(x, gw_ref[...], preferred_element_type=jnp.float32)
    logits = logits + gb_ref[...]
    iota_e = jax.lax.broadcasted_iota(jnp.int32, logits.shape, 1)
    v1 = jnp.max(logits, axis=-1, keepdims=True)
    i1 = jnp.argmax(logits, axis=-1, keepdims=True).astype(jnp.int32)
    masked = jnp.where(iota_e == i1, -jnp.inf, logits)
    v2 = jnp.max(masked, axis=-1, keepdims=True)
    i2 = jnp.argmax(masked, axis=-1, keepdims=True).astype(jnp.int32)
    t = jnp.exp(v2 - v1)
    w_first = 1.0 / (1.0 + t)
    w_second = t / (1.0 + t)
    idx_ref[...] = jnp.concatenate([i1, i2], axis=1)
    w_ref[...] = jnp.concatenate([w_first, w_second], axis=1)


def _gating(xf, gate_W, gb2):
    T = xf.shape[0]
    return pl.pallas_call(
        _gate_body,
        out_shape=[
            jax.ShapeDtypeStruct((T, TOP_K), jnp.int32),
            jax.ShapeDtypeStruct((T, TOP_K), jnp.float32),
        ],
    )(xf, gate_W, gb2)


def _sc_dispatch(xf, ridx3, tw3, R):
    T = xf.shape[0]
    info = plsc.get_sparse_core_info()
    NC, NS = info.num_cores, info.num_subcores
    NW = NC * NS
    tpw = T // NW
    mesh = plsc.VectorSubcoreMesh(core_axis_name="c", subcore_axis_name="s")

    @functools.partial(
        pl.kernel, mesh=mesh,
        out_type=[
            jax.ShapeDtypeStruct((R, MODEL_DIM), jnp.float32),
            jax.ShapeDtypeStruct((R,), jnp.float32),
        ],
        scratch_types=[
            pltpu.VMEM((TOP_K, tpw), jnp.int32),
            pltpu.VMEM((TOP_K, tpw), jnp.float32),
            pltpu.VMEM((tpw, MODEL_DIM), jnp.float32),
            pltpu.SemaphoreType.DMA,
            pltpu.SemaphoreType.DMA,
        ],
    )
    def dispatch_k(src_hbm, idx_hbm, tw_hbm, out_hbm, wc_hbm,
                   idx_v, tw_v, rows_v, lsem, ssem):
        wid = lax.axis_index("s") * NC + lax.axis_index("c")
        pltpu.sync_copy(idx_hbm.at[wid], idx_v)
        pltpu.sync_copy(tw_hbm.at[wid], tw_v)
        ld = pltpu.async_copy(
            src_hbm.at[pl.ds(wid * tpw, tpw)], rows_v, lsem)
        ld.wait()
        scs = []
        for k in range(TOP_K):
            scs.append(
                pltpu.async_copy(rows_v, out_hbm.at[idx_v.at[k]], ssem))
            scs.append(
                pltpu.async_copy(tw_v.at[k], wc_hbm.at[idx_v.at[k]], ssem))
        for sc in scs:
            sc.wait()

    return dispatch_k(xf, ridx3, tw3)


def _sc_return_combine(yw, ridx3, T):
    info = plsc.get_sparse_core_info()
    NC, NS = info.num_cores, info.num_subcores
    NW = NC * NS
    tpw = T // NW
    n16 = MODEL_DIM // LANES
    mesh = plsc.VectorSubcoreMesh(core_axis_name="c", subcore_axis_name="s")

    @functools.partial(
        pl.kernel, mesh=mesh,
        out_type=jax.ShapeDtypeStruct((T, MODEL_DIM), jnp.float32),
        scratch_types=[
            pltpu.VMEM((TOP_K, tpw), jnp.int32),
            pltpu.VMEM((tpw, MODEL_DIM), jnp.float32),
            pltpu.VMEM((tpw, MODEL_DIM), jnp.float32),
            pltpu.SemaphoreType.DMA,
            pltpu.SemaphoreType.DMA,
            pltpu.SemaphoreType.DMA,
        ],
    )
    def ret_k(rows_hbm, idx_hbm, out_hbm, idx_v, buf0, buf1, g0, g1, ssem):
        wid = lax.axis_index("s") * NC + lax.axis_index("c")
        pltpu.sync_copy(idx_hbm.at[wid], idx_v)
        ga = pltpu.async_copy(rows_hbm.at[idx_v.at[0]], buf0, g0)
        gb = pltpu.async_copy(rows_hbm.at[idx_v.at[1]], buf1, g1)
        ga.wait()
        gb.wait()

        def add_row(r, _):
            for j in range(n16):
                sl = pl.ds(j * LANES, LANES)
                buf0[r, sl] = buf0[r, sl] + buf1[r, sl]
            return 0

        lax.fori_loop(0, tpw, add_row, 0)
        st = pltpu.async_copy(
            buf0, out_hbm.at[pl.ds(wid * tpw, tpw)], ssem)
        st.wait()

    return ret_k(yw, ridx3)


def _ffn_body(be_ref, bv_ref, xg_ref, w1_ref, b1_ref, w2_ref, b2_ref, wc_ref,
              out_ref):
    del be_ref

    @pl.when(bv_ref[pl.program_id(0)] == 1)
    def _():
        xb = xg_ref[...]
        h = (jnp.dot(xb, w1_ref[0], preferred_element_type=jnp.float32)
             + b1_ref[0])
        h = jnp.maximum(h, 0.0)
        y = (jnp.dot(h, w2_ref[0], preferred_element_type=jnp.float32)
             + b2_ref[0])
        out_ref[...] = y * wc_ref[...]


def _grouped_ffn(xg, blk_exp, blk_valid, W1, b1r, W2, b2r, wc, NB):
    R = xg.shape[0]
    grid_spec = pltpu.PrefetchScalarGridSpec(
        num_scalar_prefetch=2,
        grid=(NB,),
        in_specs=[
            pl.BlockSpec((BT, MODEL_DIM), lambda i, be, bv: (i, 0)),
            pl.BlockSpec((1, MODEL_DIM, DIM_FF),
                         lambda i, be, bv: (be[i], 0, 0)),
            pl.BlockSpec((1, 1, DIM_FF), lambda i, be, bv: (be[i], 0, 0)),
            pl.BlockSpec((1, DIM_FF, MODEL_DIM),
                         lambda i, be, bv: (be[i], 0, 0)),
            pl.BlockSpec((1, 1, MODEL_DIM), lambda i, be, bv: (be[i], 0, 0)),
            pl.BlockSpec((BT, 1), lambda i, be, bv: (i, 0)),
        ],
        out_specs=pl.BlockSpec((BT, MODEL_DIM), lambda i, be, bv: (i, 0)),
    )
    return pl.pallas_call(
        _ffn_body,
        grid_spec=grid_spec,
        out_shape=jax.ShapeDtypeStruct((R, MODEL_DIM), jnp.float32),
        compiler_params=pltpu.CompilerParams(
            dimension_semantics=("arbitrary",),
        ),
    )(blk_exp, blk_valid, xg, W1, b1r, W2, b2r, wc)


def kernel(x, gate_W, gate_b, W1, b1, W2, b2):
    batch, seq, _ = x.shape
    xf = x.reshape(-1, MODEL_DIM)
    T = xf.shape[0]
    P = T * TOP_K
    NB = P // BT + NUM_EXPERTS
    R = NB * BT
    NW = 32
    tpw = T // NW
    gb2 = gate_b.reshape(1, NUM_EXPERTS)
    b1r = b1.reshape(NUM_EXPERTS, 1, DIM_FF)
    b2r = b2.reshape(NUM_EXPERTS, 1, MODEL_DIM)

    top_idx, top_w = _gating(xf, gate_W, gb2)

    e_flat = top_idx.reshape(-1)
    eids = jnp.arange(NUM_EXPERTS, dtype=jnp.int32)
    oh = (e_flat[:, None] == eids[None, :]).astype(jnp.int32)
    csum = jnp.cumsum(oh, axis=0)
    counts = csum[-1]
    rank = jnp.sum(jnp.where(oh == 1, csum, 0), axis=1) - 1
    nb_e = (counts + BT - 1) // BT
    first_blk = jnp.cumsum(nb_e) - nb_e
    end_blk = first_blk + nb_e
    blk = jnp.arange(NB)
    total_blk = jnp.sum(nb_e)
    blk_valid = (blk < total_blk).astype(jnp.int32)
    blk_exp = jnp.sum(blk[:, None] >= end_blk[None, :], axis=1)
    last_exp = jnp.max(jnp.where(counts > 0, eids, 0))
    blk_exp = jnp.where(blk_valid == 1,
                        jnp.minimum(blk_exp, NUM_EXPERTS - 1),
                        last_exp).astype(jnp.int32)
    fb_p = jnp.sum(jnp.where(oh == 1, first_blk[None, :], 0), axis=1)
    r_of_p = (fb_p + rank // BT) * BT + rank % BT
    r_of_p = r_of_p.astype(jnp.int32)
    ridx3 = r_of_p.reshape(NW, tpw, TOP_K).transpose(0, 2, 1)
    tw3 = top_w.reshape(NW, tpw, TOP_K).transpose(0, 2, 1)

    xg, row_w = _sc_dispatch(xf, ridx3, tw3, R)
    wc = row_w.reshape(R, 1)
    yw = _grouped_ffn(xg, blk_exp, blk_valid, W1, b1r, W2, b2r, wc, NB)
    out = _sc_return_combine(yw, ridx3, T)
    return out.reshape(batch, seq, MODEL_DIM)


# speedup vs baseline: 1.0892x; 1.0892x over previous
import functools

import jax
import jax.numpy as jnp
from jax import lax
from jax.experimental import pallas as pl
from jax.experimental.pallas import tpu as pltpu
from jax.experimental.pallas import tpu_sc as plsc

MODEL_DIM = 768
DIM_FF = 2048
NUM_EXPERTS = 8
TOP_K = 2

BT = 256
LANES = 16


def _gate_body(xf_ref, gw_ref, gb_ref, idx_ref, w_ref):
    x = xf_ref[...]
    logits = jnp.dot(x, gw_ref[...], preferred_element_type=jnp.float32)
    logits = logits + gb_ref[...]
    iota_e = jax.lax.broadcasted_iota(jnp.int32, logits.shape, 1)
    v1 = jnp.max(logits, axis=-1, keepdims=True)
    i1 = jnp.argmax(logits, axis=-1, keepdims=True).astype(jnp.int32)
    masked = jnp.where(iota_e == i1, -jnp.inf, logits)
    v2 = jnp.max(masked, axis=-1, keepdims=True)
    i2 = jnp.argmax(masked, axis=-1, keepdims=True).astype(jnp.int32)
    t = jnp.exp(v2 - v1)
    w_first = 1.0 / (1.0 + t)
    w_second = t / (1.0 + t)
    idx_ref[...] = jnp.concatenate([i1, i2], axis=1)
    w_ref[...] = jnp.concatenate([w_first, w_second], axis=1)


def _gating(xf, gate_W, gb2):
    T = xf.shape[0]
    return pl.pallas_call(
        _gate_body,
        out_shape=[
            jax.ShapeDtypeStruct((T, TOP_K), jnp.int32),
            jax.ShapeDtypeStruct((T, TOP_K), jnp.float32),
        ],
    )(xf, gate_W, gb2)


def _sc_dispatch(xf, ridx3, R):
    T = xf.shape[0]
    info = plsc.get_sparse_core_info()
    NC, NS = info.num_cores, info.num_subcores
    NW = NC * NS
    tpw = T // NW
    mesh = plsc.VectorSubcoreMesh(core_axis_name="c", subcore_axis_name="s")

    @functools.partial(
        pl.kernel, mesh=mesh,
        out_type=jax.ShapeDtypeStruct((R, MODEL_DIM), jnp.float32),
        scratch_types=[
            pltpu.VMEM((TOP_K, tpw), jnp.int32),
            pltpu.VMEM((tpw, MODEL_DIM), jnp.float32),
            pltpu.SemaphoreType.DMA,
            pltpu.SemaphoreType.DMA,
        ],
    )
    def dispatch_k(src_hbm, idx_hbm, out_hbm, idx_v, rows_v, lsem, ssem):
        wid = lax.axis_index("s") * NC + lax.axis_index("c")
        pltpu.sync_copy(idx_hbm.at[wid], idx_v)
        ld = pltpu.async_copy(
            src_hbm.at[pl.ds(wid * tpw, tpw)], rows_v, lsem)
        ld.wait()
        scs = []
        for k in range(TOP_K):
            scs.append(
                pltpu.async_copy(rows_v, out_hbm.at[idx_v.at[k]], ssem))
        for sc in scs:
            sc.wait()

    return dispatch_k(xf, ridx3)


def _sc_return_combine(yw, ridx3, T):
    info = plsc.get_sparse_core_info()
    NC, NS = info.num_cores, info.num_subcores
    NW = NC * NS
    tpw = T // NW
    n16 = MODEL_DIM // LANES
    mesh = plsc.VectorSubcoreMesh(core_axis_name="c", subcore_axis_name="s")

    @functools.partial(
        pl.kernel, mesh=mesh,
        out_type=jax.ShapeDtypeStruct((T, MODEL_DIM), jnp.float32),
        scratch_types=[
            pltpu.VMEM((TOP_K, tpw), jnp.int32),
            pltpu.VMEM((tpw, MODEL_DIM), jnp.float32),
            pltpu.VMEM((tpw, MODEL_DIM), jnp.float32),
            pltpu.SemaphoreType.DMA,
            pltpu.SemaphoreType.DMA,
            pltpu.SemaphoreType.DMA,
        ],
    )
    def ret_k(rows_hbm, idx_hbm, out_hbm, idx_v, buf0, buf1, g0, g1, ssem):
        wid = lax.axis_index("s") * NC + lax.axis_index("c")
        pltpu.sync_copy(idx_hbm.at[wid], idx_v)
        ga = pltpu.async_copy(rows_hbm.at[idx_v.at[0]], buf0, g0)
        gb = pltpu.async_copy(rows_hbm.at[idx_v.at[1]], buf1, g1)
        ga.wait()
        gb.wait()

        def add_row(r, _):
            for j in range(n16):
                sl = pl.ds(j * LANES, LANES)
                buf0[r, sl] = buf0[r, sl] + buf1[r, sl]
            return 0

        lax.fori_loop(0, tpw, add_row, 0)
        st = pltpu.async_copy(
            buf0, out_hbm.at[pl.ds(wid * tpw, tpw)], ssem)
        st.wait()

    return ret_k(yw, ridx3)


def _ffn_body(be_ref, bv_ref, xg_ref, w1_ref, b1_ref, w2_ref, b2_ref, wc_ref,
              out_ref):
    del be_ref

    @pl.when(bv_ref[pl.program_id(0)] == 1)
    def _():
        xb = xg_ref[...]
        h = (jnp.dot(xb, w1_ref[0], preferred_element_type=jnp.float32)
             + b1_ref[0])
        h = jnp.maximum(h, 0.0)
        y = (jnp.dot(h, w2_ref[0], preferred_element_type=jnp.float32)
             + b2_ref[0])
        out_ref[...] = y * wc_ref[...]


def _grouped_ffn(xg, blk_exp, blk_valid, W1, b1r, W2, b2r, wc, NB):
    R = xg.shape[0]
    grid_spec = pltpu.PrefetchScalarGridSpec(
        num_scalar_prefetch=2,
        grid=(NB,),
        in_specs=[
            pl.BlockSpec((BT, MODEL_DIM), lambda i, be, bv: (i, 0)),
            pl.BlockSpec((1, MODEL_DIM, DIM_FF),
                         lambda i, be, bv: (be[i], 0, 0)),
            pl.BlockSpec((1, 1, DIM_FF), lambda i, be, bv: (be[i], 0, 0)),
            pl.BlockSpec((1, DIM_FF, MODEL_DIM),
                         lambda i, be, bv: (be[i], 0, 0)),
            pl.BlockSpec((1, 1, MODEL_DIM), lambda i, be, bv: (be[i], 0, 0)),
            pl.BlockSpec((BT, 1), lambda i, be, bv: (i, 0)),
        ],
        out_specs=pl.BlockSpec((BT, MODEL_DIM), lambda i, be, bv: (i, 0)),
    )
    return pl.pallas_call(
        _ffn_body,
        grid_spec=grid_spec,
        out_shape=jax.ShapeDtypeStruct((R, MODEL_DIM), jnp.float32),
        compiler_params=pltpu.CompilerParams(
            dimension_semantics=("arbitrary",),
        ),
    )(blk_exp, blk_valid, xg, W1, b1r, W2, b2r, wc)


def kernel(x, gate_W, gate_b, W1, b1, W2, b2):
    batch, seq, _ = x.shape
    xf = x.reshape(-1, MODEL_DIM)
    T = xf.shape[0]
    P = T * TOP_K
    NB = P // BT + NUM_EXPERTS
    R = NB * BT
    NW = 32
    tpw = T // NW
    gb2 = gate_b.reshape(1, NUM_EXPERTS)
    b1r = b1.reshape(NUM_EXPERTS, 1, DIM_FF)
    b2r = b2.reshape(NUM_EXPERTS, 1, MODEL_DIM)

    top_idx, top_w = _gating(xf, gate_W, gb2)

    e_flat = top_idx.reshape(-1)
    eids = jnp.arange(NUM_EXPERTS, dtype=jnp.int32)
    oh = (e_flat[:, None] == eids[None, :]).astype(jnp.int32)
    csum = jnp.cumsum(oh, axis=0)
    counts = csum[-1]
    rank = jnp.sum(jnp.where(oh == 1, csum, 0), axis=1) - 1
    nb_e = (counts + BT - 1) // BT
    first_blk = jnp.cumsum(nb_e) - nb_e
    end_blk = first_blk + nb_e
    blk = jnp.arange(NB)
    total_blk = jnp.sum(nb_e)
    blk_valid = (blk < total_blk).astype(jnp.int32)
    blk_exp = jnp.sum(blk[:, None] >= end_blk[None, :], axis=1)
    last_exp = jnp.max(jnp.where(counts > 0, eids, 0))
    blk_exp = jnp.where(blk_valid == 1,
                        jnp.minimum(blk_exp, NUM_EXPERTS - 1),
                        last_exp).astype(jnp.int32)
    fb_p = jnp.sum(jnp.where(oh == 1, first_blk[None, :], 0), axis=1)
    r_of_p = (fb_p + rank // BT) * BT + rank % BT
    r_of_p = r_of_p.astype(jnp.int32)
    ridx3 = r_of_p.reshape(NW, tpw, TOP_K).transpose(0, 2, 1)
    row_w = jnp.zeros((R,), jnp.float32).at[r_of_p].set(top_w.reshape(-1))
    wc = row_w.reshape(R, 1)

    xg = _sc_dispatch(xf, ridx3, R)
    yw = _grouped_ffn(xg, blk_exp, blk_valid, W1, b1r, W2, b2r, wc, NB)
    out = _sc_return_combine(yw, ridx3, T)
    return out.reshape(batch, seq, MODEL_DIM)
